# baseline (speedup 1.0000x reference)
import functools

import jax
import jax.numpy as jnp
from jax import lax
from jax.experimental import pallas as pl
from jax.experimental.pallas import tpu as pltpu
from jax.experimental.pallas import tpu_sc as plsc

N, C, H, W = 8, 32, 512, 512
R = N * H
NW = 32
L = 16
HB = 8
WQ = 256
CH = 16
DUMP = CH
NBLK = R // HB
BLKS_PER_W = NBLK // NW
CHUNKS = HB * WQ // L
NBUF = 2


def _onehot_body(x_hbm, out_hbm, xbuf, clsbuf, obuf,
                 in_sem0, in_sem1, out_sem0, out_sem1):
    cid = lax.axis_index("c")
    sid = lax.axis_index("s")
    wid = sid * 2 + cid
    blk0 = wid * BLKS_PER_W

    iota = lax.iota(jnp.int32, L)
    ones_f = jnp.full((L,), 1.0, dtype=jnp.float32)
    zeros_f = jnp.zeros((L,), dtype=jnp.float32)
    zeros_i = jnp.zeros((L,), dtype=jnp.int32)

    in_sems = (in_sem0, in_sem1)
    out_sems = (out_sem0, out_sem1)

    pltpu.async_copy(
        x_hbm.at[pl.ds(pl.multiple_of(blk0 * HB, HB), HB), :],
        xbuf.at[0], in_sem0)

    def zero_row(c, carry):
        for b in range(NBUF):
            for hh in range(HB):
                for j in range(WQ // L):
                    obuf[b, c, hh, pl.ds(j * L, L)] = zeros_f
        return carry
    lax.fori_loop(0, CH + 1, zero_row, 0)
    for b in range(NBUF):
        for j in range(CHUNKS):
            clsbuf[b, pl.ds(j * L, L)] = zeros_i

    def outer(b, carry):
        blk = blk0 + b
        r = blk * HB
        n = lax.shift_right_logical(r, 9)
        h = pl.multiple_of(lax.bitwise_and(r, H - 1), HB)
        xb = lax.rem(b, 2)

        @pl.when(xb == 0)
        def _wait_in0():
            pltpu.make_async_copy(
                x_hbm.at[pl.ds(pl.multiple_of(blk * HB, HB), HB), :],
                xbuf.at[0], in_sems[0]).wait()

        @pl.when(xb == 1)
        def _wait_in1():
            pltpu.make_async_copy(
                x_hbm.at[pl.ds(pl.multiple_of(blk * HB, HB), HB), :],
                xbuf.at[1], in_sems[1]).wait()

        @pl.when(jnp.logical_and(b < BLKS_PER_W - 1, xb == 0))
        def _prefetch1():
            pltpu.async_copy(
                x_hbm.at[pl.ds(pl.multiple_of((blk + 1) * HB, HB), HB), :],
                xbuf.at[1], in_sems[1])

        @pl.when(jnp.logical_and(b < BLKS_PER_W - 1, xb == 1))
        def _prefetch0():
            pltpu.async_copy(
                x_hbm.at[pl.ds(pl.multiple_of((blk + 1) * HB, HB), HB), :],
                xbuf.at[0], in_sems[0])

        for jj in range(4):
            c0 = (jj // 2) * CH
            w0 = (jj % 2) * WQ
            p = jj % NBUF

            @pl.when(jnp.logical_or(b > 0, jj >= NBUF))
            def _wait_out():
                pltpu.make_async_copy(
                    obuf.at[p, pl.ds(0, CH)],
                    out_hbm.at[n, pl.ds(c0, CH), pl.ds(h, HB), pl.ds(w0, WQ)],
                    out_sems[p]).wait()

            for ch in range(CHUNKS):
                hh = ch // (WQ // L)
                wl = (ch % (WQ // L)) * L
                jv = iota + wl
                hv = jnp.full((L,), hh, dtype=jnp.int32)
                cv = clsbuf[p, pl.ds(ch * L, L)]
                plsc.store_scatter(obuf.at[p], [cv, hv, jv], zeros_f)
            for ch in range(CHUNKS):
                hh = ch // (WQ // L)
                wl = (ch % (WQ // L)) * L
                jv = iota + wl
                hv = jnp.full((L,), hh, dtype=jnp.int32)
                xv = xbuf[xb, hh, pl.ds(w0 + wl, L)]
                lo = xv & (CH - 1)
                if c0 == 0:
                    in_half = xv < CH
                else:
                    in_half = xv >= CH
                cv = jnp.where(in_half, lo, DUMP)
                plsc.store_scatter(obuf.at[p], [cv, hv, jv], ones_f)
                clsbuf[p, pl.ds(ch * L, L)] = cv

            pltpu.async_copy(
                obuf.at[p, pl.ds(0, CH)],
                out_hbm.at[n, pl.ds(c0, CH), pl.ds(h, HB), pl.ds(w0, WQ)],
                out_sems[p])
        return carry

    lax.fori_loop(0, BLKS_PER_W, outer, 0)

    blk = blk0 + BLKS_PER_W - 1
    r = blk * HB
    n = lax.shift_right_logical(r, 9)
    h = pl.multiple_of(lax.bitwise_and(r, H - 1), HB)
    for jj in range(2, 4):
        c0 = (jj // 2) * CH
        w0 = (jj % 2) * WQ
        p = jj % NBUF
        pltpu.make_async_copy(
            obuf.at[p, pl.ds(0, CH)],
            out_hbm.at[n, pl.ds(c0, CH), pl.ds(h, HB), pl.ds(w0, WQ)],
            out_sems[p]).wait()


@jax.jit
def _onehot_sc(x2):
    mesh = plsc.VectorSubcoreMesh(core_axis_name="c", subcore_axis_name="s")
    return pl.kernel(
        _onehot_body,
        mesh=mesh,
        compiler_params=pltpu.CompilerParams(needs_layout_passes=False),
        out_type=jax.ShapeDtypeStruct((N, C, H, W), jnp.float32),
        scratch_types=[
            pltpu.VMEM((NBUF, HB, W), jnp.int32),
            pltpu.VMEM((NBUF, CHUNKS * L), jnp.int32),
            pltpu.VMEM((NBUF, CH + 1, HB, WQ), jnp.float32),
            pltpu.SemaphoreType.DMA,
            pltpu.SemaphoreType.DMA,
            pltpu.SemaphoreType.DMA,
            pltpu.SemaphoreType.DMA,
        ],
    )(x2)


def kernel(x):
    x2 = x.reshape(R, W).astype(jnp.int32)
    return _onehot_sc(x2)


# speedup vs baseline: 1.9028x; 1.9028x over previous
import functools

import jax
import jax.numpy as jnp
from jax import lax
from jax.experimental import pallas as pl
from jax.experimental.pallas import tpu as pltpu
from jax.experimental.pallas import tpu_sc as plsc

N, C, H, W = 8, 32, 512, 512
R = N * H
NW = 32
L = 16
HB = 8
WQ = 128
NJOBS_BLK = W // WQ
NBLK = R // HB
BLKS_PER_W = NBLK // NW
CHUNKS = HB * WQ // L
NBUF = 2


def _onehot_body(x_hbm, out_hbm, xbuf, clsbuf, obuf,
                 in_sem0, in_sem1, out_sem0, out_sem1):
    cid = lax.axis_index("c")
    sid = lax.axis_index("s")
    wid = sid * 2 + cid
    blk0 = wid * BLKS_PER_W

    iota = lax.iota(jnp.int32, L)
    ones_f = jnp.full((L,), 1.0, dtype=jnp.float32)
    zeros_f = jnp.zeros((L,), dtype=jnp.float32)
    zeros_i = jnp.zeros((L,), dtype=jnp.int32)

    in_sems = (in_sem0, in_sem1)
    out_sems = (out_sem0, out_sem1)

    pltpu.async_copy(
        x_hbm.at[pl.ds(pl.multiple_of(blk0 * HB, HB), HB), :],
        xbuf.at[0], in_sem0)

    def zero_row(c, carry):
        for b in range(NBUF):
            for hh in range(HB):
                for j in range(WQ // L):
                    obuf[b, c, hh, pl.ds(j * L, L)] = zeros_f
        return carry
    lax.fori_loop(0, C, zero_row, 0)
    for b in range(NBUF):
        for j in range(HB * WQ // L):
            clsbuf[b, pl.ds(j * L, L)] = zeros_i

    def outer(b, carry):
        blk = blk0 + b
        r = blk * HB
        n = lax.shift_right_logical(r, 9)
        h = pl.multiple_of(lax.bitwise_and(r, H - 1), HB)
        xb = lax.rem(b, 2)

        @pl.when(xb == 0)
        def _wait_in0():
            pltpu.make_async_copy(
                x_hbm.at[pl.ds(pl.multiple_of(blk * HB, HB), HB), :],
                xbuf.at[0], in_sems[0]).wait()

        @pl.when(xb == 1)
        def _wait_in1():
            pltpu.make_async_copy(
                x_hbm.at[pl.ds(pl.multiple_of(blk * HB, HB), HB), :],
                xbuf.at[1], in_sems[1]).wait()

        @pl.when(jnp.logical_and(b < BLKS_PER_W - 1, xb == 0))
        def _prefetch1():
            pltpu.async_copy(
                x_hbm.at[pl.ds(pl.multiple_of((blk + 1) * HB, HB), HB), :],
                xbuf.at[1], in_sems[1])

        @pl.when(jnp.logical_and(b < BLKS_PER_W - 1, xb == 1))
        def _prefetch0():
            pltpu.async_copy(
                x_hbm.at[pl.ds(pl.multiple_of((blk + 1) * HB, HB), HB), :],
                xbuf.at[0], in_sems[0])

        for jj in range(NJOBS_BLK):
            w0 = jj * WQ
            p = jj % NBUF

            @pl.when(jnp.logical_or(b > 0, jj >= NBUF))
            def _wait_out():
                pltpu.make_async_copy(
                    obuf.at[p],
                    out_hbm.at[n, :, pl.ds(h, HB), pl.ds(w0, WQ)],
                    out_sems[p]).wait()

            for ch in range(CHUNKS):
                hh = ch // (WQ // L)
                wl = (ch % (WQ // L)) * L
                jv = iota + wl
                hv = jnp.full((L,), hh, dtype=jnp.int32)
                cv = clsbuf[p, pl.ds(ch * L, L)]
                plsc.store_scatter(obuf.at[p], [cv, hv, jv], zeros_f)
                xv = xbuf[xb, hh, pl.ds(w0 + wl, L)]
                plsc.store_scatter(obuf.at[p], [xv, hv, jv], ones_f)
                clsbuf[p, pl.ds(ch * L, L)] = xv

            pltpu.async_copy(
                obuf.at[p],
                out_hbm.at[n, :, pl.ds(h, HB), pl.ds(w0, WQ)],
                out_sems[p])
        return carry

    lax.fori_loop(0, BLKS_PER_W, outer, 0)

    blk = blk0 + BLKS_PER_W - 1
    r = blk * HB
    n = lax.shift_right_logical(r, 9)
    h = pl.multiple_of(lax.bitwise_and(r, H - 1), HB)
    for jj in range(NJOBS_BLK - NBUF, NJOBS_BLK):
        w0 = jj * WQ
        p = jj % NBUF
        pltpu.make_async_copy(
            obuf.at[p],
            out_hbm.at[n, :, pl.ds(h, HB), pl.ds(w0, WQ)],
            out_sems[p]).wait()


@jax.jit
def _onehot_sc(x2):
    mesh = plsc.VectorSubcoreMesh(core_axis_name="c", subcore_axis_name="s")
    return pl.kernel(
        _onehot_body,
        mesh=mesh,
        compiler_params=pltpu.CompilerParams(needs_layout_passes=False),
        out_type=jax.ShapeDtypeStruct((N, C, H, W), jnp.float32),
        scratch_types=[
            pltpu.VMEM((NBUF, HB, W), jnp.int32),
            pltpu.VMEM((NBUF, HB * WQ), jnp.int32),
            pltpu.VMEM((NBUF, C, HB, WQ), jnp.float32),
            pltpu.SemaphoreType.DMA,
            pltpu.SemaphoreType.DMA,
            pltpu.SemaphoreType.DMA,
            pltpu.SemaphoreType.DMA,
        ],
    )(x2)


def kernel(x):
    x2 = x.reshape(R, W).astype(jnp.int32)
    return _onehot_sc(x2)


# speedup vs baseline: 1.9098x; 1.0036x over previous
import jax
import jax.numpy as jnp
from jax import lax
from jax.experimental import pallas as pl
from jax.experimental.pallas import tpu as pltpu
from jax.experimental.pallas import tpu_sc as plsc

N, C, H, W = 8, 32, 512, 512
R = N * H
NW = 32
L = 16
HB = 8
WQ = 128
NJOBS_BLK = W // WQ
NBLK = R // HB
BLKS_PER_W = NBLK // NW
CHUNKS = HB * WQ // L
NBUF = 2


def _onehot_body(x_hbm, out_hbm, xbuf, clsbuf, obuf,
                 in_sem0, in_sem1, out_sem0, out_sem1, out_sem2, out_sem3):
    cid = lax.axis_index("c")
    sid = lax.axis_index("s")
    wid = sid * 2 + cid
    blk0 = wid * BLKS_PER_W

    iota = lax.iota(jnp.int32, L)
    ones_f = jnp.full((L,), 1.0, dtype=jnp.float32)
    zeros_f = jnp.zeros((L,), dtype=jnp.float32)
    zeros_i = jnp.zeros((L,), dtype=jnp.int32)

    in_sems = (in_sem0, in_sem1)
    out_sems = ((out_sem0, out_sem1), (out_sem2, out_sem3))

    pltpu.async_copy(
        x_hbm.at[pl.ds(pl.multiple_of(blk0 * HB, HB), HB), :],
        xbuf.at[0], in_sem0)

    def zero_row(c, carry):
        for b in range(NBUF):
            for hh in range(HB):
                for j in range(WQ // L):
                    obuf[b, c, hh, pl.ds(j * L, L)] = zeros_f
        return carry
    lax.fori_loop(0, C, zero_row, 0)
    for b in range(NBUF):
        for j in range(HB * WQ // L):
            clsbuf[b, pl.ds(j * L, L)] = zeros_i

    def outer(b, carry):
        blk = blk0 + b
        r = blk * HB
        n = lax.shift_right_logical(r, 9)
        h = pl.multiple_of(lax.bitwise_and(r, H - 1), HB)
        xb = lax.rem(b, 2)

        @pl.when(xb == 0)
        def _wait_in0():
            pltpu.make_async_copy(
                x_hbm.at[pl.ds(pl.multiple_of(blk * HB, HB), HB), :],
                xbuf.at[0], in_sems[0]).wait()

        @pl.when(xb == 1)
        def _wait_in1():
            pltpu.make_async_copy(
                x_hbm.at[pl.ds(pl.multiple_of(blk * HB, HB), HB), :],
                xbuf.at[1], in_sems[1]).wait()

        @pl.when(jnp.logical_and(b < BLKS_PER_W - 1, xb == 0))
        def _prefetch1():
            pltpu.async_copy(
                x_hbm.at[pl.ds(pl.multiple_of((blk + 1) * HB, HB), HB), :],
                xbuf.at[1], in_sems[1])

        @pl.when(jnp.logical_and(b < BLKS_PER_W - 1, xb == 1))
        def _prefetch0():
            pltpu.async_copy(
                x_hbm.at[pl.ds(pl.multiple_of((blk + 1) * HB, HB), HB), :],
                xbuf.at[0], in_sems[0])

        for jj in range(NJOBS_BLK):
            w0 = jj * WQ
            p = jj % NBUF

            @pl.when(jnp.logical_or(b > 0, jj >= NBUF))
            def _wait_out():
                for half in range(2):
                    pltpu.make_async_copy(
                        obuf.at[p, pl.ds(half * (C // 2), C // 2)],
                        out_hbm.at[n, pl.ds(half * (C // 2), C // 2),
                                   pl.ds(h, HB), pl.ds(w0, WQ)],
                        out_sems[p][half]).wait()

            for ch in range(CHUNKS):
                hh = ch // (WQ // L)
                wl = (ch % (WQ // L)) * L
                jv = iota + wl
                hv = jnp.full((L,), hh, dtype=jnp.int32)
                cv = clsbuf[p, pl.ds(ch * L, L)]
                plsc.store_scatter(obuf.at[p], [cv, hv, jv], zeros_f)
                xv = xbuf[xb, hh, pl.ds(w0 + wl, L)]
                plsc.store_scatter(obuf.at[p], [xv, hv, jv], ones_f)
                clsbuf[p, pl.ds(ch * L, L)] = xv

            for half in range(2):
                pltpu.async_copy(
                    obuf.at[p, pl.ds(half * (C // 2), C // 2)],
                    out_hbm.at[n, pl.ds(half * (C // 2), C // 2),
                               pl.ds(h, HB), pl.ds(w0, WQ)],
                    out_sems[p][half])
        return carry

    lax.fori_loop(0, BLKS_PER_W, outer, 0)

    blk = blk0 + BLKS_PER_W - 1
    r = blk * HB
    n = lax.shift_right_logical(r, 9)
    h = pl.multiple_of(lax.bitwise_and(r, H - 1), HB)
    for jj in range(NJOBS_BLK - NBUF, NJOBS_BLK):
        w0 = jj * WQ
        p = jj % NBUF
        for half in range(2):
            pltpu.make_async_copy(
                obuf.at[p, pl.ds(half * (C // 2), C // 2)],
                out_hbm.at[n, pl.ds(half * (C // 2), C // 2),
                           pl.ds(h, HB), pl.ds(w0, WQ)],
                out_sems[p][half]).wait()


@jax.jit
def _onehot_sc(x2):
    mesh = plsc.VectorSubcoreMesh(core_axis_name="c", subcore_axis_name="s")
    return pl.kernel(
        _onehot_body,
        mesh=mesh,
        compiler_params=pltpu.CompilerParams(needs_layout_passes=False),
        out_type=jax.ShapeDtypeStruct((N, C, H, W), jnp.float32),
        scratch_types=[
            pltpu.VMEM((NBUF, HB, W), jnp.int32),
            pltpu.VMEM((NBUF, HB * WQ), jnp.int32),
            pltpu.VMEM((NBUF, C, HB, WQ), jnp.float32),
            pltpu.SemaphoreType.DMA,
            pltpu.SemaphoreType.DMA,
            pltpu.SemaphoreType.DMA,
            pltpu.SemaphoreType.DMA,
            pltpu.SemaphoreType.DMA,
            pltpu.SemaphoreType.DMA,
        ],
    )(x2)


def kernel(x):
    x2 = x.reshape(R, W).astype(jnp.int32)
    return _onehot_sc(x2)
